# baseline (speedup 1.0000x reference)
import functools

import jax
import jax.numpy as jnp
from jax import lax
from jax.experimental import pallas as pl
from jax.experimental.pallas import tpu as pltpu
from jax.experimental.pallas import tpu_sc as plsc

VOCAB = 1_000_000
D = 64

_info = plsc.get_sparse_core_info()
NC = _info.num_cores
NS = _info.num_subcores
NW = NC * NS

CHUNK = 128


def _make_kernel(B):
    assert B % NW == 0
    b_per_w = B // NW
    assert b_per_w % CHUNK == 0
    ncnk = b_per_w // CHUNK

    mesh = plsc.VectorSubcoreMesh(core_axis_name="c", subcore_axis_name="s")

    @functools.partial(
        pl.kernel,
        out_type=jax.ShapeDtypeStruct((B, D), jnp.float32),
        mesh=mesh,
        scratch_types=[
            pltpu.VMEM((ncnk, CHUNK), jnp.int32),
            pltpu.VMEM((CHUNK, D), jnp.float32),
            pltpu.SemaphoreType.DMA,
        ],
        compiler_params=pltpu.CompilerParams(use_tc_tiling_on_sc=False),
    )
    def emb(idx_hbm, table_hbm, out_hbm, idx_v, rows_v, gsem):
        wid = lax.axis_index("s") * NC + lax.axis_index("c")
        base = wid * b_per_w

        pltpu.sync_copy(idx_hbm.at[wid], idx_v)

        def body(g, _):
            pltpu.async_copy(table_hbm.at[idx_v.at[g]], rows_v, gsem).wait()
            pltpu.sync_copy(rows_v, out_hbm.at[pl.ds(base + g * CHUNK, CHUNK)])
            return 0

        lax.fori_loop(0, ncnk, body, 0)

    return emb


@jax.jit
def kernel(token_ids, weight):
    shape = token_ids.shape
    B = 1
    for s in shape:
        B *= s
    idx = token_ids.reshape(NW, B // (NW * CHUNK), CHUNK).astype(jnp.int32)
    out = _make_kernel(B)(idx, weight)
    return out.reshape(*shape, D)


# speedup vs baseline: 1.1161x; 1.1161x over previous
import functools

import jax
import jax.numpy as jnp
from jax import lax
from jax.experimental import pallas as pl
from jax.experimental.pallas import tpu as pltpu
from jax.experimental.pallas import tpu_sc as plsc

VOCAB = 1_000_000
D = 64

_info = plsc.get_sparse_core_info()
NC = _info.num_cores
NS = _info.num_subcores
NW = NC * NS

CHUNK = 128
NBUF = 8
K = 4


def _make_kernel(B):
    assert B % NW == 0
    b_per_w = B // NW
    assert b_per_w % CHUNK == 0
    ncnk = b_per_w // CHUNK
    T = ncnk // NBUF
    assert ncnk % NBUF == 0 and T >= 2

    mesh = plsc.VectorSubcoreMesh(core_axis_name="c", subcore_axis_name="s")

    @functools.partial(
        pl.kernel,
        out_type=jax.ShapeDtypeStruct((B, D), jnp.float32),
        mesh=mesh,
        scratch_types=[
            pltpu.VMEM((ncnk, CHUNK), jnp.int32),
            pltpu.VMEM((NBUF, CHUNK, D), jnp.float32),
        ]
        + [pltpu.SemaphoreType.DMA] * (2 * NBUF),
        compiler_params=pltpu.CompilerParams(use_tc_tiling_on_sc=False),
    )
    def emb(idx_hbm, table_hbm, out_hbm, idx_v, rows_v, *sems):
        gsems = sems[:NBUF]
        osems = sems[NBUF:]
        wid = lax.axis_index("s") * NC + lax.axis_index("c")
        base = wid * b_per_w

        pltpu.sync_copy(idx_hbm.at[wid], idx_v)

        def start_gather(g, s):
            pltpu.async_copy(table_hbm.at[idx_v.at[g]], rows_v.at[s], gsems[s])

        def wait_gather(s):
            pltpu.make_async_copy(
                table_hbm.at[idx_v.at[0]], rows_v.at[s], gsems[s]
            ).wait()

        def start_out(g, s):
            pltpu.async_copy(
                rows_v.at[s], out_hbm.at[pl.ds(base + g * CHUNK, CHUNK)], osems[s]
            )

        def wait_out(s):
            pltpu.make_async_copy(
                rows_v.at[s], out_hbm.at[pl.ds(0, CHUNK)], osems[s]
            ).wait()

        def visit(g, b, first=False, last=False):
            s2 = (b + K) % NBUF
            if (not first) or (b >= NBUF - K):
                wait_out(s2)
            if (not last) or (b < NBUF - K):
                start_gather(g + K, s2)
            wait_gather(b)
            start_out(g, b)

        for c in range(K):
            start_gather(c, c)

        for b in range(NBUF):
            visit(b, b, first=True)

        def outer(t, _):
            for b in range(NBUF):
                visit(t * NBUF + b, b)
            return 0

        lax.fori_loop(1, T - 1, outer, 0)

        for b in range(NBUF):
            visit(ncnk - NBUF + b, b, last=True)
        for g in range(ncnk - (NBUF - K), ncnk):
            wait_out(g % NBUF)

    return emb


@jax.jit
def kernel(token_ids, weight):
    shape = token_ids.shape
    B = 1
    for s in shape:
        B *= s
    idx = token_ids.reshape(NW, B // (NW * CHUNK), CHUNK).astype(jnp.int32)
    out = _make_kernel(B)(idx, weight)
    return out.reshape(*shape, D)


# speedup vs baseline: 1.4835x; 1.3291x over previous
import functools

import jax
import jax.numpy as jnp
from jax import lax
from jax.experimental import pallas as pl
from jax.experimental.layout import Layout, with_layout_constraint
from jax.experimental.pallas import tpu as pltpu
from jax.experimental.pallas import tpu_sc as plsc

VOCAB = 1_000_000
D = 64
DP = 128

_info = plsc.get_sparse_core_info()
NC = _info.num_cores
NS = _info.num_subcores
NW = NC * NS

CHUNK = 128
NBUF = 5
K = 3


def _make_kernel(B):
    assert B % NW == 0
    b_per_w = B // NW
    assert b_per_w % CHUNK == 0
    ncnk = b_per_w // CHUNK
    T = ncnk // NBUF
    assert ncnk % NBUF == 0 and T >= 2

    mesh = plsc.VectorSubcoreMesh(core_axis_name="c", subcore_axis_name="s")

    @functools.partial(
        pl.kernel,
        out_type=jax.ShapeDtypeStruct((B, DP), jnp.float32),
        mesh=mesh,
        scratch_types=[
            pltpu.VMEM((ncnk, CHUNK), jnp.int32),
            pltpu.VMEM((NBUF, CHUNK, D), jnp.float32),
        ]
        + [pltpu.SemaphoreType.DMA] * (2 * NBUF),
        compiler_params=pltpu.CompilerParams(use_tc_tiling_on_sc=False),
    )
    def emb(idx_hbm, table_hbm, out_hbm, idx_v, rows_v, *sems):
        gsems = sems[:NBUF]
        osems = sems[NBUF:]
        wid = lax.axis_index("s") * NC + lax.axis_index("c")
        base = wid * b_per_w

        pltpu.sync_copy(idx_hbm.at[wid], idx_v)

        def start_gather(g, s):
            pltpu.async_copy(table_hbm.at[idx_v.at[g]], rows_v.at[s], gsems[s])

        def wait_gather(s):
            pltpu.make_async_copy(
                table_hbm.at[idx_v.at[0]], rows_v.at[s], gsems[s]
            ).wait()

        def start_out(g, s):
            pltpu.async_copy(
                rows_v.at[s],
                out_hbm.at[pl.ds(base + g * CHUNK, CHUNK), pl.ds(0, D)],
                osems[s],
            )

        def wait_out(s):
            pltpu.make_async_copy(
                rows_v.at[s],
                out_hbm.at[pl.ds(0, CHUNK), pl.ds(0, D)],
                osems[s],
            ).wait()

        def visit(g, b, first=False, last=False):
            s2 = (b + K) % NBUF
            if (not first) or (b >= NBUF - K):
                wait_out(s2)
            if (not last) or (b < NBUF - K):
                start_gather(g + K, s2)
            wait_gather(b)
            start_out(g, b)

        for c in range(K):
            start_gather(c, c)

        for b in range(NBUF):
            visit(b, b, first=True)

        def outer(t, _):
            for b in range(NBUF):
                visit(t * NBUF + b, b)
            return 0

        lax.fori_loop(1, T - 1, outer, 0)

        for b in range(NBUF):
            visit(ncnk - NBUF + b, b, last=True)
        for g in range(ncnk - (NBUF - K), ncnk):
            wait_out(g % NBUF)

    return emb


@jax.jit
def kernel(token_ids, weight):
    shape = token_ids.shape
    B = 1
    for s in shape:
        B *= s
    idx = token_ids.reshape(NW, B // (NW * CHUNK), CHUNK).astype(jnp.int32)
    out = _make_kernel(B)(idx, weight)
    out = out[:, :D].reshape(*shape, D)
    return with_layout_constraint(out, Layout((2, 1, 0)))
